# baseline (speedup 1.0000x reference)
import functools

import jax
import jax.numpy as jnp
from jax import lax
from jax.experimental import pallas as pl
from jax.experimental.pallas import tpu as pltpu
from jax.experimental.pallas import tpu_sc as plsc

N = 10000
E = 320000
D = 128
NUM_LAYERS = 3

NC = 2
NS = 16
NW = NC * NS

CHUNK = 128
TOTC = 2560
EPAD = TOTC * CHUNK
CPT0 = 160
CPT1 = 0
PB = 80
K = 2

NPAD = 10240
RPT = NPAD // NS

@functools.cache
def _mesh():
    return plsc.VectorSubcoreMesh(
        core_axis_name="c", subcore_axis_name="s", num_cores=NC, num_subcores=NS
    )


def _deg_body(pidx_hbm, degp_hbm, pidx, dbuf, ones_v, zbuf, dacc):
    cid = lax.axis_index("c")
    sid = lax.axis_index("s")
    wid = cid * NS + sid
    base = wid * (TOTC // NW)

    z16 = jnp.zeros((16,), jnp.float32)
    o16 = jnp.ones((16,), jnp.float32)
    for j in range(CHUNK // 16):
        ones_v[pl.ds(j * 16, 16)] = o16
    for j in range(RPT // 16):
        zbuf[pl.ds(j * 16, 16)] = z16

    pltpu.sync_copy(zbuf, dacc.at[pl.ds(sid * RPT, RPT)])
    pltpu.sync_copy(pidx_hbm.at[pl.ds(base, TOTC // NW)], pidx)
    plsc.subcore_barrier()

    def step(c, carry):
        for j in range(CHUNK // 16):
            v = pidx[c, pl.ds(j * 16, 16)]
            dbuf[pl.ds(j * 16, 16)] = lax.shift_right_logical(v, 14)
        pltpu.sync_copy(ones_v, dacc.at[dbuf], add=True)
        return carry

    lax.fori_loop(0, TOTC // NW, step, 0)
    plsc.subcore_barrier()
    pltpu.sync_copy(dacc.at[pl.ds(sid * RPT, RPT)],
                    degp_hbm.at[cid, pl.ds(sid * RPT, RPT)])


@jax.jit
def _deg_call(pidx_p):
    return pl.kernel(
        _deg_body,
        out_type=jax.ShapeDtypeStruct((NC, NPAD), jnp.float32),
        mesh=_mesh(),
        scratch_types=[
            pltpu.VMEM((TOTC // NW, CHUNK), jnp.int32),
            pltpu.VMEM((CHUNK,), jnp.int32),
            pltpu.VMEM((CHUNK,), jnp.float32),
            pltpu.VMEM((RPT,), jnp.float32),
            pltpu.VMEM_SHARED((NPAD,), jnp.float32),
        ],
    )(pidx_p)


def _layer_body(t_hbm, pidx_hbm, sp_hbm, pidx, sidx_r, didx_r, rows,
                acc, gsems, ssems):
    cid = lax.axis_index("c")
    sid = lax.axis_index("s")

    n_c = jnp.where(cid == 0, CPT0, CPT1)
    cbase = pl.multiple_of(
        jnp.where(cid == 0, sid * CPT0, NS * CPT0 + sid * CPT1), 8)

    z16 = jnp.zeros((16,), jnp.float32)

    def zrow(r, carry):
        for j in range(D // 16):
            rows[0, r, pl.ds(j * 16, 16)] = z16
        return carry

    lax.fori_loop(0, CHUNK, zrow, 0)
    for b in range(RPT // CHUNK):
        pltpu.sync_copy(rows.at[0], acc.at[pl.ds(sid * RPT + b * CHUNK, CHUNK)])
    pltpu.sync_copy(pidx_hbm.at[pl.ds(cbase, PB)], pidx)
    plsc.subcore_barrier()

    def unpack(l, b):
        for j in range(CHUNK // 16):
            v = pidx[l % PB, pl.ds(j * 16, 16)]
            sidx_r[b, pl.ds(j * 16, 16)] = v & 0x3FFF
            didx_r[b, pl.ds(j * 16, 16)] = lax.shift_right_logical(v, 14)

    def fire_gather(l, b):
        @pl.when(jnp.logical_and(l % PB == 0, l > 0))
        def _():
            pltpu.sync_copy(
                pidx_hbm.at[pl.ds(pl.multiple_of(cbase + l, 8), PB)], pidx)

        unpack(l, b)
        pltpu.async_copy(t_hbm.at[sidx_r.at[b]], rows.at[b], gsems.at[b])

    def drain_gather(b):
        pltpu.make_async_copy(t_hbm.at[sidx_r.at[b]], rows.at[b],
                              gsems.at[b]).wait()

    def fire_scatter(b):
        pltpu.async_copy(rows.at[b], acc.at[didx_r.at[b]], ssems.at[b],
                         add=True)

    def drain_scatter(b):
        pltpu.make_async_copy(rows.at[b], acc.at[pl.ds(0, CHUNK)],
                              ssems.at[b]).wait()

    def prologue(b, carry):
        fire_gather(b, b)
        return carry

    lax.fori_loop(0, jnp.minimum(K - 1, n_c), prologue, 0)

    def step(l, carry):
        b_fire = (l + K - 1) % K

        @pl.when(l > 0)
        def _():
            drain_scatter(b_fire)

        @pl.when(l + K - 1 < n_c)
        def _():
            fire_gather(l + K - 1, b_fire)

        b = l % K
        drain_gather(b)
        fire_scatter(b)
        return carry

    lax.fori_loop(0, n_c, step, 0)

    @pl.when(n_c > 0)
    def _():
        drain_scatter((n_c - 1) % K)
    plsc.subcore_barrier()
    pltpu.sync_copy(acc.at[pl.ds(sid * RPT, RPT)],
                    sp_hbm.at[cid, pl.ds(sid * RPT, RPT)])


@jax.jit
def _layer_call(t, pidx_p):
    return pl.kernel(
        _layer_body,
        out_type=jax.ShapeDtypeStruct((NC, NPAD, D), jnp.float32),
        mesh=_mesh(),
        scratch_types=[
            pltpu.VMEM((PB, CHUNK), jnp.int32),
            pltpu.VMEM((K, CHUNK), jnp.int32),
            pltpu.VMEM((K, CHUNK), jnp.int32),
            pltpu.VMEM((K, CHUNK, D), jnp.float32),
            pltpu.VMEM_SHARED((NPAD, D), jnp.float32),
            pltpu.SemaphoreType.DMA((K,)),
            pltpu.SemaphoreType.DMA((K,)),
        ],
    )(t, pidx_p)


def _scales_body(degp_ref, dis_ref, d2_ref):
    deg = degp_ref[0] + degp_ref[1]
    dis = jnp.where(deg > 0, lax.rsqrt(deg), 0.0)
    dis_ref[...] = dis
    d2_ref[...] = dis * dis


@jax.jit
def _scales_call(degp3):
    return pl.pallas_call(
        _scales_body,
        out_shape=(
            jax.ShapeDtypeStruct((NPAD // 128, 128), jnp.float32),
            jax.ShapeDtypeStruct((NPAD // 128, 128), jnp.float32),
        ),
    )(degp3)


def _rowscale_body(x_ref, s_ref, o_ref):
    o_ref[...] = x_ref[...] * s_ref[...]


@jax.jit
def _rowscale_call(x_p, dis_c):
    blk = 2048
    return pl.pallas_call(
        _rowscale_body,
        grid=(NPAD // blk,),
        in_specs=[
            pl.BlockSpec((blk, D), lambda i: (i, 0)),
            pl.BlockSpec((blk, 1), lambda i: (i, 0)),
        ],
        out_specs=pl.BlockSpec((blk, D), lambda i: (i, 0)),
        out_shape=jax.ShapeDtypeStruct((NPAD, D), jnp.float32),
    )(x_p, dis_c)


def _mid_body(sp_ref, d2_ref, prev_ref, t_ref, ssum_ref):
    s = sp_ref[0] + sp_ref[1]
    ssum_ref[...] = prev_ref[...] + s
    t_ref[...] = s * d2_ref[...]


@jax.jit
def _mid_call(sp, d2_c, prev):
    blk = 2048
    return pl.pallas_call(
        _mid_body,
        grid=(NPAD // blk,),
        in_specs=[
            pl.BlockSpec((NC, blk, D), lambda i: (0, i, 0)),
            pl.BlockSpec((blk, 1), lambda i: (i, 0)),
            pl.BlockSpec((blk, D), lambda i: (i, 0)),
        ],
        out_specs=[
            pl.BlockSpec((blk, D), lambda i: (i, 0)),
            pl.BlockSpec((blk, D), lambda i: (i, 0)),
        ],
        out_shape=(
            jax.ShapeDtypeStruct((NPAD, D), jnp.float32),
            jax.ShapeDtypeStruct((NPAD, D), jnp.float32),
        ),
    )(sp, d2_c, prev)


def _scaleout_body(ssum_ref, dis_ref, o_ref):
    alpha = 1.0 / (1.0 + NUM_LAYERS)
    o_ref[...] = ssum_ref[...] * (dis_ref[...] * alpha)


@jax.jit
def _scaleout_call(ssum, dis_c):
    blk = 2048
    return pl.pallas_call(
        _scaleout_body,
        grid=(NPAD // blk,),
        in_specs=[
            pl.BlockSpec((blk, D), lambda i: (i, 0)),
            pl.BlockSpec((blk, 1), lambda i: (i, 0)),
        ],
        out_specs=pl.BlockSpec((blk, D), lambda i: (i, 0)),
        out_shape=jax.ShapeDtypeStruct((NPAD, D), jnp.float32),
    )(ssum, dis_c)


def kernel(x, edge_index):
    src = edge_index[0]
    dst = edge_index[1]
    pad = EPAD - E
    fill = jnp.full((pad,), NPAD - 1, jnp.int32)
    src_p = jnp.concatenate([src.astype(jnp.int32), fill])
    dst_p = jnp.concatenate([dst.astype(jnp.int32), fill])
    dummy = jnp.full((PB * CHUNK,), (NPAD - 1) | ((NPAD - 1) << 14), jnp.int32)
    pidx_p = jnp.concatenate([src_p | (dst_p << 14), dummy]).reshape(
        TOTC + PB, CHUNK)
    x_p = jnp.zeros((NPAD, D), jnp.float32).at[:N].set(x)

    degp = _deg_call(pidx_p)
    dis80, d280 = _scales_call(degp.reshape(NC, NPAD // 128, 128))
    dis_c = dis80.reshape(NPAD, 1)
    d2_c = d280.reshape(NPAD, 1)

    t0 = _rowscale_call(x_p, dis_c)
    prev0 = jnp.zeros((NPAD, D), jnp.float32)

    def body(_, carry):
        t, prev = carry
        sp = _layer_call(t, pidx_p)
        return _mid_call(sp, d2_c, prev)

    _, ssum = lax.fori_loop(0, NUM_LAYERS, body, (t0, prev0))
    out_p = _scaleout_call(ssum, dis_c)
    return out_p[:N]


# speedup vs baseline: 2.3193x; 2.3193x over previous
import functools

import jax
import jax.numpy as jnp
from jax import lax
from jax.experimental import pallas as pl
from jax.experimental.pallas import tpu as pltpu
from jax.experimental.pallas import tpu_sc as plsc

N = 10000
E = 320000
D = 128
NUM_LAYERS = 3

NC = 2
NS = 16
NW = NC * NS

CHUNK = 128
TOTC = 2560
EPAD = TOTC * CHUNK
CPT0 = 136
CPT1 = 24
PB = 80
K = 2

NPAD = 10240
RPT = NPAD // NS

@functools.cache
def _mesh():
    return plsc.VectorSubcoreMesh(
        core_axis_name="c", subcore_axis_name="s", num_cores=NC, num_subcores=NS
    )


def _deg_body(pidx_hbm, degp_hbm, pidx, dbuf, ones_v, zbuf, dacc):
    cid = lax.axis_index("c")
    sid = lax.axis_index("s")
    wid = cid * NS + sid
    base = wid * (TOTC // NW)

    z16 = jnp.zeros((16,), jnp.float32)
    o16 = jnp.ones((16,), jnp.float32)
    for j in range(CHUNK // 16):
        ones_v[pl.ds(j * 16, 16)] = o16
    for j in range(RPT // 16):
        zbuf[pl.ds(j * 16, 16)] = z16

    pltpu.sync_copy(zbuf, dacc.at[pl.ds(sid * RPT, RPT)])
    pltpu.sync_copy(pidx_hbm.at[pl.ds(base, TOTC // NW)], pidx)
    plsc.subcore_barrier()

    def step(c, carry):
        for j in range(CHUNK // 16):
            v = pidx[c, pl.ds(j * 16, 16)]
            dbuf[pl.ds(j * 16, 16)] = lax.shift_right_logical(v, 14)
        pltpu.sync_copy(ones_v, dacc.at[dbuf], add=True)
        return carry

    lax.fori_loop(0, TOTC // NW, step, 0)
    plsc.subcore_barrier()
    pltpu.sync_copy(dacc.at[pl.ds(sid * RPT, RPT)],
                    degp_hbm.at[cid, pl.ds(sid * RPT, RPT)])


@jax.jit
def _deg_call(pidx_p):
    return pl.kernel(
        _deg_body,
        out_type=jax.ShapeDtypeStruct((NC, NPAD), jnp.float32),
        mesh=_mesh(),
        scratch_types=[
            pltpu.VMEM((TOTC // NW, CHUNK), jnp.int32),
            pltpu.VMEM((CHUNK,), jnp.int32),
            pltpu.VMEM((CHUNK,), jnp.float32),
            pltpu.VMEM((RPT,), jnp.float32),
            pltpu.VMEM_SHARED((NPAD,), jnp.float32),
        ],
    )(pidx_p)


def _layer_body(t_hbm, pidx_hbm, sp_hbm, pidx, sidx_r, didx_r, rows,
                acc, gsems, ssems):
    cid = lax.axis_index("c")
    sid = lax.axis_index("s")

    n_c = jnp.where(cid == 0, CPT0, CPT1)
    cbase = pl.multiple_of(
        jnp.where(cid == 0, sid * CPT0, NS * CPT0 + sid * CPT1), 8)

    z16 = jnp.zeros((16,), jnp.float32)

    def zrow(r, carry):
        for j in range(D // 16):
            rows[0, r, pl.ds(j * 16, 16)] = z16
        return carry

    lax.fori_loop(0, CHUNK, zrow, 0)
    for b in range(RPT // CHUNK):
        pltpu.sync_copy(rows.at[0], acc.at[pl.ds(sid * RPT + b * CHUNK, CHUNK)])
    pltpu.sync_copy(pidx_hbm.at[pl.ds(cbase, PB)], pidx)
    plsc.subcore_barrier()

    def unpack(l, b):
        for j in range(CHUNK // 16):
            v = pidx[l % PB, pl.ds(j * 16, 16)]
            sidx_r[b, pl.ds(j * 16, 16)] = v & 0x3FFF
            didx_r[b, pl.ds(j * 16, 16)] = lax.shift_right_logical(v, 14)

    def fire_gather(l, b):
        @pl.when(jnp.logical_and(l % PB == 0, l > 0))
        def _():
            pltpu.sync_copy(
                pidx_hbm.at[pl.ds(pl.multiple_of(cbase + l, 8), PB)], pidx)

        unpack(l, b)
        pltpu.async_copy(t_hbm.at[sidx_r.at[b]], rows.at[b], gsems.at[b])

    def drain_gather(b):
        pltpu.make_async_copy(t_hbm.at[sidx_r.at[b]], rows.at[b],
                              gsems.at[b]).wait()

    def fire_scatter(b):
        pltpu.async_copy(rows.at[b], acc.at[didx_r.at[b]], ssems.at[b],
                         add=True)

    def drain_scatter(b):
        pltpu.make_async_copy(rows.at[b], acc.at[pl.ds(0, CHUNK)],
                              ssems.at[b]).wait()

    def prologue(b, carry):
        fire_gather(b, b)
        return carry

    lax.fori_loop(0, jnp.minimum(K - 1, n_c), prologue, 0)

    def step(l, carry):
        b_fire = (l + K - 1) % K

        @pl.when(l > 0)
        def _():
            drain_scatter(b_fire)

        @pl.when(l + K - 1 < n_c)
        def _():
            fire_gather(l + K - 1, b_fire)

        b = l % K
        drain_gather(b)
        fire_scatter(b)
        return carry

    lax.fori_loop(0, n_c, step, 0)

    @pl.when(n_c > 0)
    def _():
        drain_scatter((n_c - 1) % K)
    plsc.subcore_barrier()
    pltpu.sync_copy(acc.at[pl.ds(sid * RPT, RPT)],
                    sp_hbm.at[cid, pl.ds(sid * RPT, RPT)])


@jax.jit
def _layer_call(t, pidx_p):
    return pl.kernel(
        _layer_body,
        out_type=jax.ShapeDtypeStruct((NC, NPAD, D), jnp.float32),
        mesh=_mesh(),
        scratch_types=[
            pltpu.VMEM((PB, CHUNK), jnp.int32),
            pltpu.VMEM((K, CHUNK), jnp.int32),
            pltpu.VMEM((K, CHUNK), jnp.int32),
            pltpu.VMEM((K, CHUNK, D), jnp.float32),
            pltpu.VMEM_SHARED((NPAD, D), jnp.float32),
            pltpu.SemaphoreType.DMA((K,)),
            pltpu.SemaphoreType.DMA((K,)),
        ],
    )(t, pidx_p)


def _scales_body(degp_ref, dis_ref, d2_ref):
    deg = degp_ref[0] + degp_ref[1]
    dis = jnp.where(deg > 0, lax.rsqrt(deg), 0.0)
    dis_ref[...] = dis
    d2_ref[...] = dis * dis


@jax.jit
def _scales_call(degp3):
    return pl.pallas_call(
        _scales_body,
        out_shape=(
            jax.ShapeDtypeStruct((NPAD // 128, 128), jnp.float32),
            jax.ShapeDtypeStruct((NPAD // 128, 128), jnp.float32),
        ),
    )(degp3)


def _rowscale_body(x_ref, s_ref, o_ref):
    o_ref[...] = x_ref[...] * s_ref[...]


@jax.jit
def _rowscale_call(x_p, dis_c):
    blk = 2048
    return pl.pallas_call(
        _rowscale_body,
        grid=(NPAD // blk,),
        in_specs=[
            pl.BlockSpec((blk, D), lambda i: (i, 0)),
            pl.BlockSpec((blk, 1), lambda i: (i, 0)),
        ],
        out_specs=pl.BlockSpec((blk, D), lambda i: (i, 0)),
        out_shape=jax.ShapeDtypeStruct((NPAD, D), jnp.float32),
    )(x_p, dis_c)


def _mid_body(sp_ref, d2_ref, prev_ref, t_ref, ssum_ref):
    s = sp_ref[0] + sp_ref[1]
    ssum_ref[...] = prev_ref[...] + s
    t_ref[...] = s * d2_ref[...]


@jax.jit
def _mid_call(sp, d2_c, prev):
    blk = 2048
    return pl.pallas_call(
        _mid_body,
        grid=(NPAD // blk,),
        in_specs=[
            pl.BlockSpec((NC, blk, D), lambda i: (0, i, 0)),
            pl.BlockSpec((blk, 1), lambda i: (i, 0)),
            pl.BlockSpec((blk, D), lambda i: (i, 0)),
        ],
        out_specs=[
            pl.BlockSpec((blk, D), lambda i: (i, 0)),
            pl.BlockSpec((blk, D), lambda i: (i, 0)),
        ],
        out_shape=(
            jax.ShapeDtypeStruct((NPAD, D), jnp.float32),
            jax.ShapeDtypeStruct((NPAD, D), jnp.float32),
        ),
    )(sp, d2_c, prev)


def _scaleout_body(ssum_ref, dis_ref, o_ref):
    alpha = 1.0 / (1.0 + NUM_LAYERS)
    o_ref[...] = ssum_ref[...] * (dis_ref[...] * alpha)


@jax.jit
def _scaleout_call(ssum, dis_c):
    blk = 2048
    return pl.pallas_call(
        _scaleout_body,
        grid=(NPAD // blk,),
        in_specs=[
            pl.BlockSpec((blk, D), lambda i: (i, 0)),
            pl.BlockSpec((blk, 1), lambda i: (i, 0)),
        ],
        out_specs=pl.BlockSpec((blk, D), lambda i: (i, 0)),
        out_shape=jax.ShapeDtypeStruct((NPAD, D), jnp.float32),
    )(ssum, dis_c)


def kernel(x, edge_index):
    src = edge_index[0]
    dst = edge_index[1]
    pad = EPAD - E
    fill = N + (jnp.arange(pad, dtype=jnp.int32) % (NPAD - N))
    src_p = jnp.concatenate([src.astype(jnp.int32), fill])
    dst_p = jnp.concatenate([dst.astype(jnp.int32), fill])
    dfill = N + (jnp.arange(PB * CHUNK, dtype=jnp.int32) % (NPAD - N))
    dummy = dfill | (dfill << 14)
    pidx_p = jnp.concatenate([src_p | (dst_p << 14), dummy]).reshape(
        TOTC + PB, CHUNK)
    x_p = jnp.zeros((NPAD, D), jnp.float32).at[:N].set(x)

    degp = _deg_call(pidx_p)
    dis80, d280 = _scales_call(degp.reshape(NC, NPAD // 128, 128))
    dis_c = dis80.reshape(NPAD, 1)
    d2_c = d280.reshape(NPAD, 1)

    t0 = _rowscale_call(x_p, dis_c)
    prev0 = jnp.zeros((NPAD, D), jnp.float32)

    def body(_, carry):
        t, prev = carry
        sp = _layer_call(t, pidx_p)
        return _mid_call(sp, d2_c, prev)

    _, ssum = lax.fori_loop(0, NUM_LAYERS, body, (t0, prev0))
    out_p = _scaleout_call(ssum, dis_c)
    return out_p[:N]


# speedup vs baseline: 2.9422x; 1.2685x over previous
import functools

import jax
import jax.numpy as jnp
from jax import lax
from jax.experimental import pallas as pl
from jax.experimental.pallas import tpu as pltpu
from jax.experimental.pallas import tpu_sc as plsc

N = 10000
E = 320000
D = 128
NUM_LAYERS = 3

NC = 2
NS = 16
NW = NC * NS

CHUNK = 128
TOTC = 2560
EPAD = TOTC * CHUNK
CPT0 = 96
CPT1 = 64
PB = 80
K = 2

NPAD = 10240
RPT = NPAD // NS

@functools.cache
def _mesh():
    return plsc.VectorSubcoreMesh(
        core_axis_name="c", subcore_axis_name="s", num_cores=NC, num_subcores=NS
    )


def _deg_body(pidx_hbm, degp_hbm, pidx, dbuf, ones_v, zbuf, dacc):
    cid = lax.axis_index("c")
    sid = lax.axis_index("s")
    wid = cid * NS + sid
    base = wid * (TOTC // NW)

    z16 = jnp.zeros((16,), jnp.float32)
    o16 = jnp.ones((16,), jnp.float32)
    for j in range(CHUNK // 16):
        ones_v[pl.ds(j * 16, 16)] = o16
    for j in range(RPT // 16):
        zbuf[pl.ds(j * 16, 16)] = z16

    pltpu.sync_copy(zbuf, dacc.at[pl.ds(sid * RPT, RPT)])
    pltpu.sync_copy(pidx_hbm.at[pl.ds(base, TOTC // NW)], pidx)
    plsc.subcore_barrier()

    def step(c, carry):
        for j in range(CHUNK // 16):
            v = pidx[c, pl.ds(j * 16, 16)]
            dbuf[pl.ds(j * 16, 16)] = lax.shift_right_logical(v, 14)
        pltpu.sync_copy(ones_v, dacc.at[dbuf], add=True)
        return carry

    lax.fori_loop(0, TOTC // NW, step, 0)
    plsc.subcore_barrier()
    pltpu.sync_copy(dacc.at[pl.ds(sid * RPT, RPT)],
                    degp_hbm.at[cid, pl.ds(sid * RPT, RPT)])


@jax.jit
def _deg_call(pidx_p):
    return pl.kernel(
        _deg_body,
        out_type=jax.ShapeDtypeStruct((NC, NPAD), jnp.float32),
        mesh=_mesh(),
        scratch_types=[
            pltpu.VMEM((TOTC // NW, CHUNK), jnp.int32),
            pltpu.VMEM((CHUNK,), jnp.int32),
            pltpu.VMEM((CHUNK,), jnp.float32),
            pltpu.VMEM((RPT,), jnp.float32),
            pltpu.VMEM_SHARED((NPAD,), jnp.float32),
        ],
    )(pidx_p)


def _layer_body(t_hbm, pidx_hbm, sp_hbm, pidx, sidx_r, didx_r, rows,
                acc, gsems, ssems):
    cid = lax.axis_index("c")
    sid = lax.axis_index("s")

    n_c = jnp.where(cid == 0, CPT0, CPT1)
    cbase = pl.multiple_of(
        jnp.where(cid == 0, sid * CPT0, NS * CPT0 + sid * CPT1), 8)

    z16 = jnp.zeros((16,), jnp.float32)

    def zrow(r, carry):
        for j in range(D // 16):
            rows[0, r, pl.ds(j * 16, 16)] = z16
        return carry

    lax.fori_loop(0, CHUNK, zrow, 0)
    for b in range(RPT // CHUNK):
        pltpu.sync_copy(rows.at[0], acc.at[pl.ds(sid * RPT + b * CHUNK, CHUNK)])
    pltpu.sync_copy(pidx_hbm.at[pl.ds(cbase, PB)], pidx)
    plsc.subcore_barrier()

    def unpack(l, b):
        for j in range(CHUNK // 16):
            v = pidx[l % PB, pl.ds(j * 16, 16)]
            sidx_r[b, pl.ds(j * 16, 16)] = v & 0x3FFF
            didx_r[b, pl.ds(j * 16, 16)] = lax.shift_right_logical(v, 14)

    def fire_gather(l, b):
        @pl.when(jnp.logical_and(l % PB == 0, l > 0))
        def _():
            pltpu.sync_copy(
                pidx_hbm.at[pl.ds(pl.multiple_of(cbase + l, 8), PB)], pidx)

        unpack(l, b)
        pltpu.async_copy(t_hbm.at[sidx_r.at[b]], rows.at[b], gsems.at[b])

    def drain_gather(b):
        pltpu.make_async_copy(t_hbm.at[sidx_r.at[b]], rows.at[b],
                              gsems.at[b]).wait()

    def fire_scatter(b):
        pltpu.async_copy(rows.at[b], acc.at[didx_r.at[b]], ssems.at[b],
                         add=True)

    def drain_scatter(b):
        pltpu.make_async_copy(rows.at[b], acc.at[pl.ds(0, CHUNK)],
                              ssems.at[b]).wait()

    def prologue(b, carry):
        fire_gather(b, b)
        return carry

    lax.fori_loop(0, jnp.minimum(K - 1, n_c), prologue, 0)

    def step(l, carry):
        b_fire = (l + K - 1) % K

        @pl.when(l > 0)
        def _():
            drain_scatter(b_fire)

        @pl.when(l + K - 1 < n_c)
        def _():
            fire_gather(l + K - 1, b_fire)

        b = l % K
        drain_gather(b)
        fire_scatter(b)
        return carry

    lax.fori_loop(0, n_c, step, 0)

    @pl.when(n_c > 0)
    def _():
        drain_scatter((n_c - 1) % K)
    plsc.subcore_barrier()
    pltpu.sync_copy(acc.at[pl.ds(sid * RPT, RPT)],
                    sp_hbm.at[cid, pl.ds(sid * RPT, RPT)])


@jax.jit
def _layer_call(t, pidx_p):
    return pl.kernel(
        _layer_body,
        out_type=jax.ShapeDtypeStruct((NC, NPAD, D), jnp.float32),
        mesh=_mesh(),
        scratch_types=[
            pltpu.VMEM((PB, CHUNK), jnp.int32),
            pltpu.VMEM((K, CHUNK), jnp.int32),
            pltpu.VMEM((K, CHUNK), jnp.int32),
            pltpu.VMEM((K, CHUNK, D), jnp.float32),
            pltpu.VMEM_SHARED((NPAD, D), jnp.float32),
            pltpu.SemaphoreType.DMA((K,)),
            pltpu.SemaphoreType.DMA((K,)),
        ],
    )(t, pidx_p)


def _scales_body(degp_ref, dis_ref, d2_ref):
    deg = degp_ref[0] + degp_ref[1]
    dis = jnp.where(deg > 0, lax.rsqrt(deg), 0.0)
    dis_ref[...] = dis
    d2_ref[...] = dis * dis


@jax.jit
def _scales_call(degp3):
    return pl.pallas_call(
        _scales_body,
        out_shape=(
            jax.ShapeDtypeStruct((NPAD // 128, 128), jnp.float32),
            jax.ShapeDtypeStruct((NPAD // 128, 128), jnp.float32),
        ),
    )(degp3)


def _rowscale_body(x_ref, s_ref, o_ref):
    o_ref[...] = x_ref[...] * s_ref[...]


@jax.jit
def _rowscale_call(x_p, dis_c):
    blk = 2048
    return pl.pallas_call(
        _rowscale_body,
        grid=(NPAD // blk,),
        in_specs=[
            pl.BlockSpec((blk, D), lambda i: (i, 0)),
            pl.BlockSpec((blk, 1), lambda i: (i, 0)),
        ],
        out_specs=pl.BlockSpec((blk, D), lambda i: (i, 0)),
        out_shape=jax.ShapeDtypeStruct((NPAD, D), jnp.float32),
    )(x_p, dis_c)


def _mid_body(sp_ref, d2_ref, prev_ref, t_ref, ssum_ref):
    s = sp_ref[0] + sp_ref[1]
    ssum_ref[...] = prev_ref[...] + s
    t_ref[...] = s * d2_ref[...]


@jax.jit
def _mid_call(sp, d2_c, prev):
    blk = 2048
    return pl.pallas_call(
        _mid_body,
        grid=(NPAD // blk,),
        in_specs=[
            pl.BlockSpec((NC, blk, D), lambda i: (0, i, 0)),
            pl.BlockSpec((blk, 1), lambda i: (i, 0)),
            pl.BlockSpec((blk, D), lambda i: (i, 0)),
        ],
        out_specs=[
            pl.BlockSpec((blk, D), lambda i: (i, 0)),
            pl.BlockSpec((blk, D), lambda i: (i, 0)),
        ],
        out_shape=(
            jax.ShapeDtypeStruct((NPAD, D), jnp.float32),
            jax.ShapeDtypeStruct((NPAD, D), jnp.float32),
        ),
    )(sp, d2_c, prev)


def _scaleout_body(ssum_ref, dis_ref, o_ref):
    alpha = 1.0 / (1.0 + NUM_LAYERS)
    o_ref[...] = ssum_ref[...] * (dis_ref[...] * alpha)


@jax.jit
def _scaleout_call(ssum, dis_c):
    blk = 2048
    return pl.pallas_call(
        _scaleout_body,
        grid=(NPAD // blk,),
        in_specs=[
            pl.BlockSpec((blk, D), lambda i: (i, 0)),
            pl.BlockSpec((blk, 1), lambda i: (i, 0)),
        ],
        out_specs=pl.BlockSpec((blk, D), lambda i: (i, 0)),
        out_shape=jax.ShapeDtypeStruct((NPAD, D), jnp.float32),
    )(ssum, dis_c)


def kernel(x, edge_index):
    src = edge_index[0]
    dst = edge_index[1]
    pad = EPAD - E
    fill = N + (jnp.arange(pad, dtype=jnp.int32) % (NPAD - N))
    src_p = jnp.concatenate([src.astype(jnp.int32), fill])
    dst_p = jnp.concatenate([dst.astype(jnp.int32), fill])
    dfill = N + (jnp.arange(PB * CHUNK, dtype=jnp.int32) % (NPAD - N))
    dummy = dfill | (dfill << 14)
    pidx_p = jnp.concatenate([src_p | (dst_p << 14), dummy]).reshape(
        TOTC + PB, CHUNK)
    x_p = jnp.zeros((NPAD, D), jnp.float32).at[:N].set(x)

    degp = _deg_call(pidx_p)
    dis80, d280 = _scales_call(degp.reshape(NC, NPAD // 128, 128))
    dis_c = dis80.reshape(NPAD, 1)
    d2_c = d280.reshape(NPAD, 1)

    t0 = _rowscale_call(x_p, dis_c)
    prev0 = jnp.zeros((NPAD, D), jnp.float32)

    def body(_, carry):
        t, prev = carry
        sp = _layer_call(t, pidx_p)
        return _mid_call(sp, d2_c, prev)

    _, ssum = lax.fori_loop(0, NUM_LAYERS, body, (t0, prev0))
    out_p = _scaleout_call(ssum, dis_c)
    return out_p[:N]
